# baseline (speedup 1.0000x reference)
import jax
import jax.numpy as jnp
from jax.experimental import pallas as pl

BLOCK = 8_000


def _net_block(z_ref, a_ref, c_ref, o_ref):
    zb = z_ref[...]
    d = jnp.dot(zb, a_ref[...], preferred_element_type=jnp.float32)
    o_ref[...] = jax.nn.sigmoid(d + c_ref[...])


def kernel(z, W, b):
    wd = W[0] - W[1]
    bd = b[0] - b[1]
    wdtile = jnp.tile(wd, 4)
    onehot = jax.nn.one_hot(jnp.arange(128) // 32, 4, dtype=jnp.float32)
    a_pos = wdtile[:, None] * onehot
    a = jnp.stack([a_pos, -a_pos], axis=2).reshape(128, 8)
    c = jnp.tile(jnp.stack([bd, -bd]), 4).reshape(1, 8)

    n = z.shape[0]
    zp = z.reshape(n // 4, 128)
    grid = (zp.shape[0] // BLOCK,)
    out = pl.pallas_call(
        _net_block,
        grid=grid,
        in_specs=[
            pl.BlockSpec((BLOCK, 128), lambda i: (i, 0)),
            pl.BlockSpec((128, 8), lambda i: (0, 0)),
            pl.BlockSpec((1, 8), lambda i: (0, 0)),
        ],
        out_specs=pl.BlockSpec((BLOCK, 8), lambda i: (i, 0)),
        out_shape=jax.ShapeDtypeStruct((n // 4, 8), jnp.float32),
    )(zp, a, c)
    return out.reshape(n, 2)


# speedup vs baseline: 14.2924x; 14.2924x over previous
import jax
import jax.numpy as jnp
from jax.experimental import pallas as pl

BN = 64_000
CB = BN // 128


def _net_block(zt_ref, w_ref, c_ref, o_ref):
    zb = zt_ref[...]
    d = jnp.dot(w_ref[...], zb, preferred_element_type=jnp.float32)
    dd = d.reshape(CB, 128) + c_ref[0, 0]
    sp = jax.nn.sigmoid(dd)
    o_ref[0::2, :] = sp
    o_ref[1::2, :] = 1.0 - sp


def kernel(z, W, b):
    wd = (W[0] - W[1]).reshape(1, 32)
    bd = (b[0] - b[1]).reshape(1, 1)
    n = z.shape[0]
    zt = z.T
    grid = (n // BN,)
    o = pl.pallas_call(
        _net_block,
        grid=grid,
        in_specs=[
            pl.BlockSpec((32, BN), lambda i: (0, i)),
            pl.BlockSpec((1, 32), lambda i: (0, 0)),
            pl.BlockSpec((1, 1), lambda i: (0, 0)),
        ],
        out_specs=pl.BlockSpec((2 * CB, 128), lambda i: (i, 0)),
        out_shape=jax.ShapeDtypeStruct((n // 64, 128), jnp.float32),
    )(zt, wd, bd)
    return o.reshape(n // 128, 2, 128).transpose(0, 2, 1).reshape(n, 2)
